# baseline (speedup 1.0000x reference)
import functools

import jax
import jax.numpy as jnp
from jax import lax
from jax.experimental import pallas as pl
from jax.experimental.pallas import tpu as pltpu
from jax.experimental.pallas import tpu_sc as plsc

TOPK = 2
EXP = 64
T_TOK = 4096
HID = 1024
INTER = 512
S_ROWS = T_TOK * TOPK
BT = 256
MT = S_ROWS // BT
GRID = MT + EXP - 1

NW = 32


def _routing_body(logits_ref, ids_ref, w_ref):
    l = logits_ref[...]
    ii = lax.broadcasted_iota(jnp.int32, l.shape, 1)
    m1 = jnp.max(l, axis=1, keepdims=True)
    i1 = jnp.min(jnp.where(l == m1, ii, EXP), axis=1, keepdims=True)
    lm = jnp.where(ii == i1, -jnp.inf, l)
    m2 = jnp.max(lm, axis=1, keepdims=True)
    i2 = jnp.min(jnp.where(lm == m2, ii, EXP), axis=1, keepdims=True)
    w1 = 1.0 / (1.0 + jnp.exp(m2 - m1))
    ids_ref[...] = jnp.concatenate([i1, i2], axis=1)
    w_ref[...] = jnp.concatenate([w1, 1.0 - w1], axis=1)


def _routing(router_logits):
    return pl.pallas_call(
        _routing_body,
        out_shape=(
            jax.ShapeDtypeStruct((T_TOK, TOPK), jnp.int32),
            jax.ShapeDtypeStruct((T_TOK, TOPK), jnp.float32),
        ),
    )(router_logits)


def _sc_gather(table, idx):
    rows_total, depth = idx.shape[0], table.shape[1]
    per_w = rows_total // NW
    chunk = 32
    n_chunks = per_w // chunk
    mesh = plsc.VectorSubcoreMesh(core_axis_name="c", subcore_axis_name="s")

    @functools.partial(
        pl.kernel,
        mesh=mesh,
        out_type=jax.ShapeDtypeStruct((rows_total, depth), jnp.float32),
        scratch_types=[
            pltpu.VMEM((per_w,), jnp.int32),
            pltpu.VMEM((2, chunk, depth), jnp.float32),
            pltpu.SemaphoreType.DMA,
            pltpu.SemaphoreType.DMA,
        ],
    )
    def k(table_hbm, idx_hbm, out_hbm, idx_v, rows_v, sem0, sem1):
        wid = lax.axis_index("s") * 2 + lax.axis_index("c")
        base = wid * per_w
        pltpu.sync_copy(idx_hbm.at[pl.ds(base, per_w)], idx_v)
        sems = [sem0, sem1]

        def issue(c):
            return pltpu.async_copy(
                table_hbm.at[idx_v.at[pl.ds(c * chunk, chunk)]],
                rows_v.at[c % 2], sems[c % 2])

        cps = {0: issue(0)}
        for c in range(n_chunks):
            if c + 1 < n_chunks:
                cps[c + 1] = issue(c + 1)
            cps[c].wait()
            pltpu.sync_copy(rows_v.at[c % 2],
                            out_hbm.at[pl.ds(base + c * chunk, chunk)])

    return k(table, idx)


def _sc_combine(y, pos0, pos1):
    depth = y.shape[1]
    per_w = T_TOK // NW
    chunk = 16
    n_chunks = per_w // chunk
    vpr = depth // 16
    mesh = plsc.VectorSubcoreMesh(core_axis_name="c", subcore_axis_name="s")

    @functools.partial(
        pl.kernel,
        mesh=mesh,
        out_type=jax.ShapeDtypeStruct((T_TOK, depth), jnp.float32),
        scratch_types=[
            pltpu.VMEM((per_w,), jnp.int32),
            pltpu.VMEM((per_w,), jnp.int32),
            pltpu.VMEM((2, chunk, depth), jnp.float32),
            pltpu.VMEM((2, chunk, depth), jnp.float32),
            pltpu.SemaphoreType.DMA,
            pltpu.SemaphoreType.DMA,
            pltpu.SemaphoreType.DMA,
            pltpu.SemaphoreType.DMA,
        ],
    )
    def k(y_hbm, p0_hbm, p1_hbm, out_hbm, p0_v, p1_v, buf0, buf1,
          s00, s01, s10, s11):
        wid = lax.axis_index("s") * 2 + lax.axis_index("c")
        base = wid * per_w
        pltpu.sync_copy(p0_hbm.at[pl.ds(base, per_w)], p0_v)
        pltpu.sync_copy(p1_hbm.at[pl.ds(base, per_w)], p1_v)
        sems0 = [s00, s01]
        sems1 = [s10, s11]

        def issue(c):
            b = c % 2
            sl = pl.ds(c * chunk, chunk)
            return (
                pltpu.async_copy(y_hbm.at[p0_v.at[sl]], buf0.at[b], sems0[b]),
                pltpu.async_copy(y_hbm.at[p1_v.at[sl]], buf1.at[b], sems1[b]),
            )

        cps = {0: issue(0)}
        for c in range(n_chunks):
            if c + 1 < n_chunks:
                cps[c + 1] = issue(c + 1)
            cpa, cpb = cps[c]
            cpa.wait()
            cpb.wait()
            b = c % 2

            @plsc.parallel_loop(0, chunk * vpr, unroll=8)
            def _add(i):
                j = i // vpr
                q = i % vpr
                sl = pl.ds(q * 16, 16)
                buf0[b, j, sl] = buf0[b, j, sl] + buf1[b, j, sl]
            pltpu.sync_copy(buf0.at[b],
                            out_hbm.at[pl.ds(base + c * chunk, chunk)])

    return k(y, pos0, pos1)


RING = 4


def _gmm_body(tile_s, group_s, rs_s, re_s, flag_s, slot_s, flagp2_s,
              slotp2_s, gp2_s,
              x_ref, w13_hbm, w2_hbm, b13_ref, b2_ref, wrow_ref, y_ref,
              ring13, ring2, sem13, sem2):
    v = pl.program_id(0)
    t = tile_s[v]
    rs = rs_s[v]
    re = re_s[v]
    prev_t = tile_s[jnp.maximum(v - 1, 0)]
    first = jnp.logical_or(v == 0, t != prev_t)

    def issue(f, s, g):
        @pl.when(f == 1)
        def _():
            pltpu.make_async_copy(w13_hbm.at[g], ring13.at[s],
                                  sem13.at[s]).start()
            pltpu.make_async_copy(w2_hbm.at[g], ring2.at[s],
                                  sem2.at[s]).start()

    @pl.when(v == 0)
    def _():
        issue(flag_s[0], slot_s[0], group_s[0])
        issue(flag_s[1], slot_s[1], group_s[1])
        issue(flag_s[2], slot_s[2], group_s[2])

    issue(flagp2_s[v], slotp2_s[v], gp2_s[v])

    @pl.when(flag_s[v] == 1)
    def _():
        s = slot_s[v]
        g = group_s[v]
        pltpu.make_async_copy(w13_hbm.at[g], ring13.at[s], sem13.at[s]).wait()
        pltpu.make_async_copy(w2_hbm.at[g], ring2.at[s], sem2.at[s]).wait()

    slot = slot_s[v]
    x = x_ref[...]
    dn = (((1,), (1,)), ((), ()))
    h = lax.dot_general(x, ring13[slot], dn,
                        preferred_element_type=jnp.float32)
    h = h + b13_ref[0]
    gate = h[:, :INTER]
    up = h[:, INTER:]
    act = gate * lax.logistic(gate) * up
    y = lax.dot_general(act, ring2[slot], dn,
                        preferred_element_type=jnp.float32)
    y = y + b2_ref[0]

    rows = t * BT + lax.broadcasted_iota(jnp.int32, (BT, 1), 0)
    inside = jnp.logical_and(rows >= rs, rows < re)
    scale = jnp.where(inside, wrow_ref[:, 0:1], 0.0)
    y = y * scale

    @pl.when(first)
    def _():
        y_ref[...] = y

    @pl.when(jnp.logical_not(first))
    def _():
        y_ref[...] = y_ref[...] + y


def _gmm(x_sorted, w13, w2, b13, b2, wrow_wide,
         tile_v, group_v, rs_v, re_v, flag_v, slot_v,
         flagp2_v, slotp2_v, gp2_v):
    idx_t = lambda v, *refs: (refs[0][v], 0)
    idx_g = lambda v, *refs: (refs[1][v], 0, 0)
    grid_spec = pltpu.PrefetchScalarGridSpec(
        num_scalar_prefetch=9,
        grid=(GRID,),
        in_specs=[
            pl.BlockSpec((BT, HID), idx_t),
            pl.BlockSpec(memory_space=pltpu.MemorySpace.HBM),
            pl.BlockSpec(memory_space=pltpu.MemorySpace.HBM),
            pl.BlockSpec((1, 1, 2 * INTER), idx_g),
            pl.BlockSpec((1, 1, HID), idx_g),
            pl.BlockSpec((BT, 8), idx_t),
        ],
        out_specs=pl.BlockSpec((BT, HID), idx_t),
        scratch_shapes=[
            pltpu.VMEM((RING, 2 * INTER, HID), jnp.float32),
            pltpu.VMEM((RING, HID, INTER), jnp.float32),
            pltpu.SemaphoreType.DMA((RING,)),
            pltpu.SemaphoreType.DMA((RING,)),
        ],
    )
    return pl.pallas_call(
        _gmm_body,
        grid_spec=grid_spec,
        out_shape=jax.ShapeDtypeStruct((S_ROWS, HID), jnp.float32),
        compiler_params=pltpu.CompilerParams(
            dimension_semantics=("arbitrary",),
        ),
    )(tile_v, group_v, rs_v, re_v, flag_v, slot_v,
      flagp2_v, slotp2_v, gp2_v,
      x_sorted, w13, w2,
      b13.reshape(EXP, 1, 2 * INTER), b2.reshape(EXP, 1, HID), wrow_wide)


def _metadata(topk_ids, topk_w):
    flat_e = topk_ids.reshape(-1)
    order = jnp.argsort(flat_e, stable=True).astype(jnp.int32)
    tok = order // TOPK
    wsorted = topk_w.reshape(-1)[order]
    inv = jnp.zeros((S_ROWS,), jnp.int32).at[order].set(
        jnp.arange(S_ROWS, dtype=jnp.int32))
    pos0 = inv[0::2]
    pos1 = inv[1::2]

    counts = jnp.bincount(flat_e, length=EXP)
    ends = jnp.cumsum(counts).astype(jnp.int32)
    starts = ends - counts
    first_tile = starts // BT
    last_tile = jnp.maximum(ends - 1, 0) // BT
    nvis = jnp.where(counts > 0, last_tile - first_tile + 1, 0)
    vend = jnp.cumsum(nvis).astype(jnp.int32)
    vstart = vend - nvis

    v = jnp.arange(GRID, dtype=jnp.int32)
    g = jnp.searchsorted(vend, v, side="right").astype(jnp.int32)
    gc = jnp.minimum(g, EXP - 1)
    tile_raw = first_tile[gc] + (v - vstart[gc])
    valid = v < vend[EXP - 1]
    tile_v = jnp.where(valid, tile_raw, MT - 1).astype(jnp.int32)
    group_v = gc.astype(jnp.int32)
    rs_v = jnp.where(valid, jnp.maximum(starts[gc], tile_raw * BT), 0).astype(jnp.int32)
    re_v = jnp.where(valid, jnp.minimum(ends[gc], (tile_raw + 1) * BT), 0).astype(jnp.int32)

    flag_v = jnp.concatenate(
        [jnp.ones((1,), jnp.int32),
         (group_v[1:] != group_v[:-1]).astype(jnp.int32)])
    slot_v = ((jnp.cumsum(flag_v) - 1) % RING).astype(jnp.int32)
    z3 = jnp.zeros((3,), jnp.int32)
    flagp2_v = jnp.concatenate([flag_v[3:], z3])
    slotp2_v = jnp.concatenate([slot_v[3:], z3])
    gp2_v = jnp.concatenate([group_v[3:], z3])
    return (tok, wsorted, pos0, pos1, tile_v, group_v, rs_v, re_v,
            flag_v, slot_v, flagp2_v, slotp2_v, gp2_v)


def kernel(hidden_states, router_logits, w13_weight, w2_weight, w13_bias, w2_bias):
    topk_ids, topk_w = _routing(router_logits)
    (tok, wsorted, pos0, pos1, tile_v, group_v, rs_v, re_v,
     flag_v, slot_v, flagp2_v, slotp2_v, gp2_v) = _metadata(topk_ids, topk_w)

    x_sorted = _sc_gather(hidden_states, tok)
    wrow_wide = jnp.broadcast_to(wsorted[:, None], (S_ROWS, 8))
    y = _gmm(x_sorted, w13_weight, w2_weight, w13_bias, w2_bias,
             wrow_wide, tile_v, group_v, rs_v, re_v,
             flag_v, slot_v, flagp2_v, slotp2_v, gp2_v)
    return _sc_combine(y, pos0, pos1)


# speedup vs baseline: 1.0032x; 1.0032x over previous
import functools

import jax
import jax.numpy as jnp
from jax import lax
from jax.experimental import pallas as pl
from jax.experimental.pallas import tpu as pltpu
from jax.experimental.pallas import tpu_sc as plsc

TOPK = 2
EXP = 64
T_TOK = 4096
HID = 1024
INTER = 512
S_ROWS = T_TOK * TOPK
BT = 256
MT = S_ROWS // BT
GRID = MT + EXP - 1

NW = 32


def _routing_body(logits_ref, ids_ref, w_ref):
    l = logits_ref[...]
    ii = lax.broadcasted_iota(jnp.int32, l.shape, 1)
    m1 = jnp.max(l, axis=1, keepdims=True)
    i1 = jnp.min(jnp.where(l == m1, ii, EXP), axis=1, keepdims=True)
    lm = jnp.where(ii == i1, -jnp.inf, l)
    m2 = jnp.max(lm, axis=1, keepdims=True)
    i2 = jnp.min(jnp.where(lm == m2, ii, EXP), axis=1, keepdims=True)
    w1 = 1.0 / (1.0 + jnp.exp(m2 - m1))
    ids_ref[...] = jnp.concatenate([i1, i2], axis=1)
    w_ref[...] = jnp.concatenate([w1, 1.0 - w1], axis=1)


def _routing(router_logits):
    return pl.pallas_call(
        _routing_body,
        out_shape=(
            jax.ShapeDtypeStruct((T_TOK, TOPK), jnp.int32),
            jax.ShapeDtypeStruct((T_TOK, TOPK), jnp.float32),
        ),
    )(router_logits)


def _sc_gather(table, idx):
    rows_total, depth = idx.shape[0], table.shape[1]
    per_w = rows_total // NW
    chunk = 32
    n_chunks = per_w // chunk
    mesh = plsc.VectorSubcoreMesh(core_axis_name="c", subcore_axis_name="s")

    @functools.partial(
        pl.kernel,
        mesh=mesh,
        out_type=jax.ShapeDtypeStruct((rows_total, depth), jnp.float32),
        scratch_types=[
            pltpu.VMEM((per_w,), jnp.int32),
            pltpu.VMEM((2, chunk, depth), jnp.float32),
            pltpu.SemaphoreType.DMA,
            pltpu.SemaphoreType.DMA,
        ],
    )
    def k(table_hbm, idx_hbm, out_hbm, idx_v, rows_v, sem0, sem1):
        wid = lax.axis_index("s") * 2 + lax.axis_index("c")
        base = wid * per_w
        pltpu.sync_copy(idx_hbm.at[pl.ds(base, per_w)], idx_v)
        sems = [sem0, sem1]

        def issue(c):
            return pltpu.async_copy(
                table_hbm.at[idx_v.at[pl.ds(c * chunk, chunk)]],
                rows_v.at[c % 2], sems[c % 2])

        cps = {0: issue(0)}
        for c in range(n_chunks):
            if c + 1 < n_chunks:
                cps[c + 1] = issue(c + 1)
            cps[c].wait()
            pltpu.sync_copy(rows_v.at[c % 2],
                            out_hbm.at[pl.ds(base + c * chunk, chunk)])

    return k(table, idx)


def _sc_combine(y, pos0, pos1):
    depth = y.shape[1]
    per_w = T_TOK // NW
    chunk = 16
    n_chunks = per_w // chunk
    vpr = depth // 16
    mesh = plsc.VectorSubcoreMesh(core_axis_name="c", subcore_axis_name="s")

    @functools.partial(
        pl.kernel,
        mesh=mesh,
        out_type=jax.ShapeDtypeStruct((T_TOK, depth), jnp.float32),
        scratch_types=[
            pltpu.VMEM((per_w,), jnp.int32),
            pltpu.VMEM((per_w,), jnp.int32),
            pltpu.VMEM((2, chunk, depth), jnp.float32),
            pltpu.VMEM((2, chunk, depth), jnp.float32),
            pltpu.SemaphoreType.DMA,
            pltpu.SemaphoreType.DMA,
            pltpu.SemaphoreType.DMA,
            pltpu.SemaphoreType.DMA,
        ],
    )
    def k(y_hbm, p0_hbm, p1_hbm, out_hbm, p0_v, p1_v, buf0, buf1,
          s00, s01, s10, s11):
        wid = lax.axis_index("s") * 2 + lax.axis_index("c")
        base = wid * per_w
        pltpu.sync_copy(p0_hbm.at[pl.ds(base, per_w)], p0_v)
        pltpu.sync_copy(p1_hbm.at[pl.ds(base, per_w)], p1_v)
        sems0 = [s00, s01]
        sems1 = [s10, s11]

        def issue(c):
            b = c % 2
            sl = pl.ds(c * chunk, chunk)
            return (
                pltpu.async_copy(y_hbm.at[p0_v.at[sl]], buf0.at[b], sems0[b]),
                pltpu.async_copy(y_hbm.at[p1_v.at[sl]], buf1.at[b], sems1[b]),
            )

        cps = {0: issue(0)}
        for c in range(n_chunks):
            if c + 1 < n_chunks:
                cps[c + 1] = issue(c + 1)
            cpa, cpb = cps[c]
            cpa.wait()
            cpb.wait()
            b = c % 2

            @plsc.parallel_loop(0, chunk * vpr, unroll=8)
            def _add(i):
                j = i // vpr
                q = i % vpr
                sl = pl.ds(q * 16, 16)
                buf0[b, j, sl] = buf0[b, j, sl] + buf1[b, j, sl]
            pltpu.sync_copy(buf0.at[b],
                            out_hbm.at[pl.ds(base + c * chunk, chunk)])

    return k(y, pos0, pos1)


RING = 3


def _gmm_body(tile_s, group_s, rs_s, re_s, flag_s, slot_s, flagp2_s,
              slotp2_s, gp2_s,
              x_ref, w13_hbm, w2_hbm, b13_ref, b2_ref, wrow_ref, y_ref,
              ring13, ring2, sem13, sem2):
    v = pl.program_id(0)
    t = tile_s[v]
    rs = rs_s[v]
    re = re_s[v]
    prev_t = tile_s[jnp.maximum(v - 1, 0)]
    first = jnp.logical_or(v == 0, t != prev_t)

    def issue(f, s, g):
        @pl.when(f == 1)
        def _():
            pltpu.make_async_copy(w13_hbm.at[g], ring13.at[s],
                                  sem13.at[s]).start()
            pltpu.make_async_copy(w2_hbm.at[g], ring2.at[s],
                                  sem2.at[s]).start()

    @pl.when(v == 0)
    def _():
        issue(flag_s[0], slot_s[0], group_s[0])
        issue(flag_s[1], slot_s[1], group_s[1])

    issue(flagp2_s[v], slotp2_s[v], gp2_s[v])

    @pl.when(flag_s[v] == 1)
    def _():
        s = slot_s[v]
        g = group_s[v]
        pltpu.make_async_copy(w13_hbm.at[g], ring13.at[s], sem13.at[s]).wait()
        pltpu.make_async_copy(w2_hbm.at[g], ring2.at[s], sem2.at[s]).wait()

    slot = slot_s[v]
    x = x_ref[...]
    dn = (((1,), (1,)), ((), ()))
    h = lax.dot_general(x, ring13[slot], dn,
                        preferred_element_type=jnp.float32)
    h = h + b13_ref[0]
    gate = h[:, :INTER]
    up = h[:, INTER:]
    act = gate * lax.logistic(gate) * up
    y = lax.dot_general(act, ring2[slot], dn,
                        preferred_element_type=jnp.float32)
    y = y + b2_ref[0]

    rows = t * BT + lax.broadcasted_iota(jnp.int32, (BT, 1), 0)
    inside = jnp.logical_and(rows >= rs, rows < re)
    scale = jnp.where(inside, wrow_ref[:, 0:1], 0.0)
    y = y * scale

    @pl.when(first)
    def _():
        y_ref[...] = y

    @pl.when(jnp.logical_not(first))
    def _():
        y_ref[...] = y_ref[...] + y


def _gmm(x_sorted, w13, w2, b13, b2, wrow_wide,
         tile_v, group_v, rs_v, re_v, flag_v, slot_v,
         flagp2_v, slotp2_v, gp2_v):
    idx_t = lambda v, *refs: (refs[0][v], 0)
    idx_g = lambda v, *refs: (refs[1][v], 0, 0)
    grid_spec = pltpu.PrefetchScalarGridSpec(
        num_scalar_prefetch=9,
        grid=(GRID,),
        in_specs=[
            pl.BlockSpec((BT, HID), idx_t),
            pl.BlockSpec(memory_space=pltpu.MemorySpace.HBM),
            pl.BlockSpec(memory_space=pltpu.MemorySpace.HBM),
            pl.BlockSpec((1, 1, 2 * INTER), idx_g),
            pl.BlockSpec((1, 1, HID), idx_g),
            pl.BlockSpec((BT, 8), idx_t),
        ],
        out_specs=pl.BlockSpec((BT, HID), idx_t),
        scratch_shapes=[
            pltpu.VMEM((RING, 2 * INTER, HID), jnp.float32),
            pltpu.VMEM((RING, HID, INTER), jnp.float32),
            pltpu.SemaphoreType.DMA((RING,)),
            pltpu.SemaphoreType.DMA((RING,)),
        ],
    )
    return pl.pallas_call(
        _gmm_body,
        grid_spec=grid_spec,
        out_shape=jax.ShapeDtypeStruct((S_ROWS, HID), jnp.float32),
        compiler_params=pltpu.CompilerParams(
            dimension_semantics=("arbitrary",),
        ),
    )(tile_v, group_v, rs_v, re_v, flag_v, slot_v,
      flagp2_v, slotp2_v, gp2_v,
      x_sorted, w13, w2,
      b13.reshape(EXP, 1, 2 * INTER), b2.reshape(EXP, 1, HID), wrow_wide)


def _metadata(topk_ids, topk_w):
    flat_e = topk_ids.reshape(-1)
    order = jnp.argsort(flat_e, stable=True).astype(jnp.int32)
    tok = order // TOPK
    wsorted = topk_w.reshape(-1)[order]
    inv = jnp.zeros((S_ROWS,), jnp.int32).at[order].set(
        jnp.arange(S_ROWS, dtype=jnp.int32))
    pos0 = inv[0::2]
    pos1 = inv[1::2]

    counts = jnp.bincount(flat_e, length=EXP)
    ends = jnp.cumsum(counts).astype(jnp.int32)
    starts = ends - counts
    first_tile = starts // BT
    last_tile = jnp.maximum(ends - 1, 0) // BT
    nvis = jnp.where(counts > 0, last_tile - first_tile + 1, 0)
    vend = jnp.cumsum(nvis).astype(jnp.int32)
    vstart = vend - nvis

    v = jnp.arange(GRID, dtype=jnp.int32)
    g = jnp.searchsorted(vend, v, side="right").astype(jnp.int32)
    gc = jnp.minimum(g, EXP - 1)
    tile_raw = first_tile[gc] + (v - vstart[gc])
    valid = v < vend[EXP - 1]
    tile_v = jnp.where(valid, tile_raw, MT - 1).astype(jnp.int32)
    group_v = gc.astype(jnp.int32)
    rs_v = jnp.where(valid, jnp.maximum(starts[gc], tile_raw * BT), 0).astype(jnp.int32)
    re_v = jnp.where(valid, jnp.minimum(ends[gc], (tile_raw + 1) * BT), 0).astype(jnp.int32)

    flag_v = jnp.concatenate(
        [jnp.ones((1,), jnp.int32),
         (group_v[1:] != group_v[:-1]).astype(jnp.int32)])
    slot_v = ((jnp.cumsum(flag_v) - 1) % RING).astype(jnp.int32)
    z2 = jnp.zeros((2,), jnp.int32)
    flagp2_v = jnp.concatenate([flag_v[2:], z2])
    slotp2_v = jnp.concatenate([slot_v[2:], z2])
    gp2_v = jnp.concatenate([group_v[2:], z2])
    return (tok, wsorted, pos0, pos1, tile_v, group_v, rs_v, re_v,
            flag_v, slot_v, flagp2_v, slotp2_v, gp2_v)


def kernel(hidden_states, router_logits, w13_weight, w2_weight, w13_bias, w2_bias):
    topk_ids, topk_w = _routing(router_logits)
    (tok, wsorted, pos0, pos1, tile_v, group_v, rs_v, re_v,
     flag_v, slot_v, flagp2_v, slotp2_v, gp2_v) = _metadata(topk_ids, topk_w)

    x_sorted = _sc_gather(hidden_states, tok)
    wrow_wide = jnp.broadcast_to(wsorted[:, None], (S_ROWS, 8))
    y = _gmm(x_sorted, w13_weight, w2_weight, w13_bias, w2_bias,
             wrow_wide, tile_v, group_v, rs_v, re_v,
             flag_v, slot_v, flagp2_v, slotp2_v, gp2_v)
    return _sc_combine(y, pos0, pos1)
